# baseline (speedup 1.0000x reference)
import functools

import jax
import jax.numpy as jnp
from jax import lax
from jax.experimental import pallas as pl
from jax.experimental.pallas import tpu as pltpu
from jax.experimental.pallas import tpu_sc as plsc

F, N, P, W, B = 3, 64, 131072, 50, 1024
FN = F * N
WIN = 64
NW = 32
BPT = B // NW
SLAB = P // NW


def _body(cf_hbm, pvm_hbm, idx_hbm, w_hbm,
          x_hbm, y_hbm, lw_hbm, npvm_hbm,
          idxall, wbuf, xbuf, ybuf, rbuf, lwidx, lwbuf, rowbuf, copybuf, sem):
    wid = lax.axis_index("s") * 2 + lax.axis_index("c")
    base_b = wid * BPT
    slab0 = wid * SLAB

    pltpu.sync_copy(idx_hbm, idxall.at[pl.ds(0, B)])

    nchunk = SLAB // 512

    def copy_chunk(c, carry):
        r0 = slab0 + c * 512
        pltpu.sync_copy(pvm_hbm.at[pl.ds(r0, 512)], copybuf)
        pltpu.sync_copy(copybuf, npvm_hbm.at[pl.ds(r0, 512)])
        return carry

    lax.fori_loop(0, nchunk, copy_chunk, 0)

    def scat(b, carry):
        i = idxall[pl.ds(b, 16)][0]

        @pl.when(jnp.logical_and(i >= slab0, i < slab0 + SLAB))
        def _():
            pltpu.sync_copy(w_hbm.at[pl.ds(b, 1)], rowbuf)
            pltpu.sync_copy(rowbuf, npvm_hbm.at[pl.ds(i, 1)])

        return carry

    lax.fori_loop(0, B, scat, 0)

    lwidx[pl.ds(0, 16)] = idxall[pl.ds(base_b, 16)] - 1
    lwidx[pl.ds(16, 16)] = idxall[pl.ds(base_b + 16, 16)] - 1
    pltpu.async_copy(pvm_hbm.at[lwidx], lwbuf, sem).wait()
    pltpu.sync_copy(lwbuf, lw_hbm.at[pl.ds(base_b, BPT)])

    lane = lax.iota(jnp.int32, 16)

    def per_b(j, carry):
        b = base_b + j
        i = idxall[pl.ds(b, 16)][0]
        a = jnp.minimum((i // 8) * 8, P - WIN)
        o = i - a
        pltpu.sync_copy(cf_hbm.at[:, pl.ds(a, WIN)], wbuf)

        for c in range(4):
            rows = lane + 16 * c
            cols = jnp.zeros((16,), jnp.int32) + (o + W - 1)
            d = plsc.load_gather(wbuf, [rows, cols])
            rbuf[pl.ds(16 * c, 16)] = 1.0 / d

        def per_row(r, carry2):
            rn = rbuf[pl.ds(lax.rem(r, N), 16)][0]
            rbase = r * W
            v0 = wbuf[r, pl.ds(o, 16)] * rn
            v1 = wbuf[r, pl.ds(o + 16, 16)] * rn
            v2 = wbuf[r, pl.ds(o + 32, 16)] * rn
            v3 = wbuf[r, pl.ds(o + W - 16, 16)] * rn
            xbuf[pl.ds(rbase, 16)] = v0
            xbuf[pl.ds(rbase + 16, 16)] = v1
            xbuf[pl.ds(rbase + 32, 16)] = v2
            xbuf[pl.ds(rbase + W - 16, 16)] = v3
            return carry2

        lax.fori_loop(0, FN, per_row, 0)

        for c in range(12):
            rows = lane + 16 * c
            cols = jnp.zeros((16,), jnp.int32) + (o + W)
            g = plsc.load_gather(wbuf, [rows, cols])
            rn = rbuf[pl.ds((16 * c) % N, 16)]
            ybuf[pl.ds(16 * c, 16)] = g * rn

        pltpu.sync_copy(xbuf, x_hbm.at[b])
        pltpu.sync_copy(ybuf, y_hbm.at[b])
        return carry

    lax.fori_loop(0, BPT, per_b, 0)


@jax.jit
def _sc_call(cf2, pvm, index, w):
    mesh = plsc.VectorSubcoreMesh(core_axis_name="c", subcore_axis_name="s")
    fn = pl.kernel(
        _body,
        out_type=[
            jax.ShapeDtypeStruct((B, FN * W), jnp.float32),
            jax.ShapeDtypeStruct((B, FN), jnp.float32),
            jax.ShapeDtypeStruct((B, N), jnp.float32),
            jax.ShapeDtypeStruct((P, N), jnp.float32),
        ],
        mesh=mesh,
        compiler_params=pltpu.CompilerParams(
            use_tc_tiling_on_sc=False, needs_layout_passes=False),
        scratch_types=[
            pltpu.VMEM((B + 16,), jnp.int32),
            pltpu.VMEM((FN, WIN), jnp.float32),
            pltpu.VMEM((FN * W,), jnp.float32),
            pltpu.VMEM((FN,), jnp.float32),
            pltpu.VMEM((N + 16,), jnp.float32),
            pltpu.VMEM((BPT,), jnp.int32),
            pltpu.VMEM((BPT, N), jnp.float32),
            pltpu.VMEM((1, N), jnp.float32),
            pltpu.VMEM((512, N), jnp.float32),
            pltpu.SemaphoreType.DMA,
        ],
    )
    return fn(cf2, pvm, index, w)


def kernel(coin_features, pvm, index, w):
    cf2 = coin_features.reshape(FN, P)
    xf, yf, last_w, new_pvm = _sc_call(cf2, pvm, index, w)
    X = xf.reshape(B, F, N, W)
    y = yf.reshape(B, F, N)
    return X, y, last_w, new_pvm


# speedup vs baseline: 1.0099x; 1.0099x over previous
import jax
import jax.numpy as jnp
from jax import lax
from jax.experimental import pallas as pl
from jax.experimental.pallas import tpu as pltpu
from jax.experimental.pallas import tpu_sc as plsc

F, N, P, W, B = 3, 64, 131072, 50, 1024
FN = F * N
WIN = 64
NW = 32
BPT = B // NW
NS = 32
SLAB = P // NS
SLAB_SHIFT = 12


def _sc_body(cf_hbm, idx_hbm, x_hbm, y_hbm, idxall, wbuf, xbuf, ybuf, rbuf):
    wid = lax.axis_index("s") * 2 + lax.axis_index("c")
    base_b = wid * BPT

    pltpu.sync_copy(idx_hbm, idxall.at[pl.ds(0, B)])

    lane = lax.iota(jnp.int32, 16)

    def per_b(j, carry):
        b = base_b + j
        i = idxall[pl.ds(b, 16)][0]
        a = jnp.minimum((i // 8) * 8, P - WIN)
        o = i - a
        pltpu.sync_copy(cf_hbm.at[:, pl.ds(a, WIN)], wbuf)

        for c in range(4):
            rows = lane + 16 * c
            cols = jnp.zeros((16,), jnp.int32) + (o + W - 1)
            d = plsc.load_gather(wbuf, [rows, cols])
            rbuf[pl.ds(16 * c, 16)] = 1.0 / d

        def per_row(r, carry2):
            rn = rbuf[pl.ds(lax.rem(r, N), 16)][0]
            rbase = r * W
            v0 = wbuf[r, pl.ds(o, 16)] * rn
            v1 = wbuf[r, pl.ds(o + 16, 16)] * rn
            v2 = wbuf[r, pl.ds(o + 32, 16)] * rn
            v3 = wbuf[r, pl.ds(o + W - 16, 16)] * rn
            xbuf[pl.ds(rbase, 16)] = v0
            xbuf[pl.ds(rbase + 16, 16)] = v1
            xbuf[pl.ds(rbase + 32, 16)] = v2
            xbuf[pl.ds(rbase + W - 16, 16)] = v3
            return carry2

        lax.fori_loop(0, FN, per_row, 0)

        for c in range(12):
            rows = lane + 16 * c
            cols = jnp.zeros((16,), jnp.int32) + (o + W)
            g = plsc.load_gather(wbuf, [rows, cols])
            rn = rbuf[pl.ds((16 * c) % N, 16)]
            ybuf[pl.ds(16 * c, 16)] = g * rn

        pltpu.sync_copy(xbuf, x_hbm.at[b])
        pltpu.sync_copy(ybuf, y_hbm.at[b])
        return carry

    lax.fori_loop(0, BPT, per_b, 0)


def _sc_xy(cf2, index):
    mesh = plsc.VectorSubcoreMesh(core_axis_name="c", subcore_axis_name="s")
    fn = pl.kernel(
        _sc_body,
        out_type=[
            jax.ShapeDtypeStruct((B, FN * W), jnp.float32),
            jax.ShapeDtypeStruct((B, FN), jnp.float32),
        ],
        mesh=mesh,
        compiler_params=pltpu.CompilerParams(
            use_tc_tiling_on_sc=False, needs_layout_passes=False),
        scratch_types=[
            pltpu.VMEM((B + 16,), jnp.int32),
            pltpu.VMEM((FN, WIN), jnp.float32),
            pltpu.VMEM((FN * W,), jnp.float32),
            pltpu.VMEM((FN,), jnp.float32),
            pltpu.VMEM((N + 16,), jnp.float32),
        ],
    )
    return fn(cf2, index)


def _tc_body(idx_ref, w_ref, pvm_ref, out_ref, lw_ref,
             perm_s, st_s, perm_g, st_g, cnt, cur):
    s = pl.program_id(0)

    @pl.when(s == 0)
    def _build_routing():
        for perm, st, shift_src in ((perm_s, st_s, 0), (perm_g, st_g, 1)):
            def zero(k, c):
                cnt[k] = 0
                return c
            lax.fori_loop(0, NS, zero, 0)

            def count(b, c):
                sl = (idx_ref[b] - shift_src) >> SLAB_SHIFT
                cnt[sl] = cnt[sl] + 1
                return c
            lax.fori_loop(0, B, count, 0)

            st[0] = 0

            def prefix(k, c):
                st[k + 1] = st[k] + cnt[k]
                cur[k] = st[k]
                return c
            lax.fori_loop(0, NS, prefix, 0)

            def place(b, c):
                sl = (idx_ref[b] - shift_src) >> SLAB_SHIFT
                perm[cur[sl]] = b
                cur[sl] = cur[sl] + 1
                return c
            lax.fori_loop(0, B, place, 0)

    out_ref[...] = pvm_ref[...]
    base = s * SLAB

    def scat(k, c):
        b = perm_s[k]
        out_ref[pl.ds(idx_ref[b] - base, 1), :] = w_ref[pl.ds(b, 1), :]
        return c

    lax.fori_loop(st_s[s], st_s[s + 1], scat, 0)

    def gath(k, c):
        b = perm_g[k]
        lw_ref[pl.ds(b, 1), :] = pvm_ref[pl.ds(idx_ref[b] - 1 - base, 1), :]
        return c

    lax.fori_loop(st_g[s], st_g[s + 1], gath, 0)


def _tc_pvm(pvm, index, w):
    return pl.pallas_call(
        _tc_body,
        grid=(NS,),
        in_specs=[
            pl.BlockSpec(memory_space=pltpu.SMEM),
            pl.BlockSpec((B, N), lambda i: (0, 0)),
            pl.BlockSpec((SLAB, N), lambda i: (i, 0)),
        ],
        out_specs=[
            pl.BlockSpec((SLAB, N), lambda i: (i, 0)),
            pl.BlockSpec((B, N), lambda i: (0, 0)),
        ],
        out_shape=[
            jax.ShapeDtypeStruct((P, N), jnp.float32),
            jax.ShapeDtypeStruct((B, N), jnp.float32),
        ],
        scratch_shapes=[
            pltpu.SMEM((B,), jnp.int32),
            pltpu.SMEM((NS + 1,), jnp.int32),
            pltpu.SMEM((B,), jnp.int32),
            pltpu.SMEM((NS + 1,), jnp.int32),
            pltpu.SMEM((NS,), jnp.int32),
            pltpu.SMEM((NS,), jnp.int32),
        ],
        compiler_params=pltpu.CompilerParams(
            dimension_semantics=("arbitrary",)),
    )(index, w, pvm)


def kernel(coin_features, pvm, index, w):
    cf2 = coin_features.reshape(FN, P)
    xf, yf = _sc_xy(cf2, index)
    new_pvm, last_w = _tc_pvm(pvm, index, w)
    X = xf.reshape(B, F, N, W)
    y = yf.reshape(B, F, N)
    return X, y, last_w, new_pvm


# speedup vs baseline: 1.0536x; 1.0433x over previous
import jax
import jax.numpy as jnp
from jax import lax
from jax.experimental import pallas as pl
from jax.experimental.pallas import tpu as pltpu
from jax.experimental.pallas import tpu_sc as plsc

F, N, P, W, B = 3, 64, 131072, 50, 1024
FN = F * N
WIN = 64
NW = 32
BPT = B // NW
GB = 16
WN = W * N
NS = 32
SLAB = P // NS
SLAB_SHIFT = 12


def _sc_body(cf_hbm, idx_hbm, xt_hbm, yt_hbm, idxall, wbufF, xtbuf, ybuf, rbufG):
    wid = lax.axis_index("s") * 2 + lax.axis_index("c")
    base_b = wid * BPT

    pltpu.sync_copy(idx_hbm, idxall.at[pl.ds(0, B)])

    lanev = lax.iota(jnp.int32, 16)
    wrow = lanev * N
    zeros = jnp.zeros((16,), jnp.int32)

    for g in range(2):
        grp = base_b + g * GB
        for f in range(3):
            def per_s(jj, carry, f=f, grp=grp, g=g):
                b = grp + jj
                i = idxall[pl.ds(b, 16)][0]
                a = jnp.minimum((i // 8) * 8, P - WIN)
                o = i - a
                pltpu.sync_copy(cf_hbm.at[pl.ds(f * N, N), pl.ds(a, WIN)],
                                wbufF)

                if f == 0:
                    for c in range(4):
                        d = plsc.load_gather(
                            wbufF, [lanev + 16 * c, zeros + (o + W - 1)])
                        rbufG[jj, pl.ds(16 * c, 16)] = 1.0 / d

                colv = zeros + jj

                def per_n(n, c2):
                    rn = rbufG[jj, pl.ds(n, 16)][0]
                    v0 = wbufF[n, pl.ds(o, 16)] * rn
                    v1 = wbufF[n, pl.ds(o + 16, 16)] * rn
                    v2 = wbufF[n, pl.ds(o + 32, 16)] * rn
                    v3 = wbufF[n, pl.ds(o + W - 16, 16)] * rn
                    plsc.store_scatter(xtbuf, [wrow + n, colv], v0)
                    plsc.store_scatter(xtbuf, [wrow + (16 * N + n), colv], v1)
                    plsc.store_scatter(xtbuf, [wrow + (32 * N + n), colv], v2)
                    plsc.store_scatter(
                        xtbuf, [wrow + ((W - 16) * N + n), colv], v3)
                    return c2

                lax.fori_loop(0, N, per_n, 0)

                ycol = zeros + (g * GB + jj)
                for c in range(4):
                    gv = plsc.load_gather(
                        wbufF, [lanev + 16 * c, zeros + (o + W)])
                    rn_v = rbufG[jj, pl.ds(16 * c, 16)]
                    plsc.store_scatter(
                        ybuf, [lanev + (f * N + 16 * c), ycol], gv * rn_v)
                return carry

            lax.fori_loop(0, GB, per_s, 0)
            pltpu.sync_copy(
                xtbuf, xt_hbm.at[pl.ds(f * WN, WN), pl.ds(grp, GB)])

    pltpu.sync_copy(ybuf, yt_hbm.at[:, pl.ds(base_b, BPT)])


def _sc_xy(cf2, index):
    mesh = plsc.VectorSubcoreMesh(core_axis_name="c", subcore_axis_name="s")
    fn = pl.kernel(
        _sc_body,
        out_type=[
            jax.ShapeDtypeStruct((F * WN, B), jnp.float32),
            jax.ShapeDtypeStruct((FN, B), jnp.float32),
        ],
        mesh=mesh,
        compiler_params=pltpu.CompilerParams(
            use_tc_tiling_on_sc=False, needs_layout_passes=False),
        scratch_types=[
            pltpu.VMEM((B + 16,), jnp.int32),
            pltpu.VMEM((N, WIN), jnp.float32),
            pltpu.VMEM((WN, GB), jnp.float32),
            pltpu.VMEM((FN, BPT), jnp.float32),
            pltpu.VMEM((GB, N + 16), jnp.float32),
        ],
    )
    return fn(cf2, index)


def _tc_body(idx_ref, wt_ref, pvmt_ref, outt_ref, lwt_ref,
             perm_s, st_s, perm_g, st_g, cnt, cur):
    s = pl.program_id(0)

    @pl.when(s == 0)
    def _build_routing():
        for perm, st, shift_src in ((perm_s, st_s, 0), (perm_g, st_g, 1)):
            def zero(k, c):
                cnt[k] = 0
                return c
            lax.fori_loop(0, NS, zero, 0)

            def count(b, c):
                sl = (idx_ref[b] - shift_src) >> SLAB_SHIFT
                cnt[sl] = cnt[sl] + 1
                return c
            lax.fori_loop(0, B, count, 0)

            st[0] = 0

            def prefix(k, c):
                st[k + 1] = st[k] + cnt[k]
                cur[k] = st[k]
                return c
            lax.fori_loop(0, NS, prefix, 0)

            def place(b, c):
                sl = (idx_ref[b] - shift_src) >> SLAB_SHIFT
                perm[cur[sl]] = b
                cur[sl] = cur[sl] + 1
                return c
            lax.fori_loop(0, B, place, 0)

    outt_ref[...] = pvmt_ref[...]
    base = s * SLAB

    def scat(k, c):
        b = perm_s[k]
        outt_ref[pl.ds(idx_ref[b] - base, 1), :] = wt_ref[pl.ds(b, 1), :]
        return c

    lax.fori_loop(st_s[s], st_s[s + 1], scat, 0)

    def gath(k, c):
        b = perm_g[k]
        lwt_ref[pl.ds(b, 1), :] = pvmt_ref[pl.ds(idx_ref[b] - 1 - base, 1), :]
        return c

    lax.fori_loop(st_g[s], st_g[s + 1], gath, 0)


def _tc_pvm(pvm, index, w):
    return pl.pallas_call(
        _tc_body,
        grid=(NS,),
        in_specs=[
            pl.BlockSpec(memory_space=pltpu.SMEM),
            pl.BlockSpec((B, N), lambda i: (0, 0)),
            pl.BlockSpec((SLAB, N), lambda i: (i, 0)),
        ],
        out_specs=[
            pl.BlockSpec((SLAB, N), lambda i: (i, 0)),
            pl.BlockSpec((B, N), lambda i: (0, 0)),
        ],
        out_shape=[
            jax.ShapeDtypeStruct((P, N), jnp.float32),
            jax.ShapeDtypeStruct((B, N), jnp.float32),
        ],
        scratch_shapes=[
            pltpu.SMEM((B,), jnp.int32),
            pltpu.SMEM((NS + 1,), jnp.int32),
            pltpu.SMEM((B,), jnp.int32),
            pltpu.SMEM((NS + 1,), jnp.int32),
            pltpu.SMEM((NS,), jnp.int32),
            pltpu.SMEM((NS,), jnp.int32),
        ],
        compiler_params=pltpu.CompilerParams(
            dimension_semantics=("arbitrary",)),
    )(index, w, pvm)


def kernel(coin_features, pvm, index, w):
    cf2 = coin_features.reshape(FN, P)
    xt2, yt2 = _sc_xy(cf2, index)
    new_pvm, last_w = _tc_pvm(pvm, index, w)
    X = xt2.reshape(F, W, N, B).transpose(3, 0, 2, 1)
    y = yt2.reshape(F, N, B).transpose(2, 0, 1)
    return X, y, last_w, new_pvm


# speedup vs baseline: 1.2690x; 1.2044x over previous
import jax
import jax.numpy as jnp
from jax import lax
from jax.experimental import pallas as pl
from jax.experimental.pallas import tpu as pltpu
from jax.experimental.pallas import tpu_sc as plsc

F, N, P, W, B = 3, 64, 131072, 50, 1024
FN = F * N
WIN = 64
NW = 32
BPT = B // NW
GB = 16
WN = W * N
NS = 32
SLAB = P // NS
SLAB_SHIFT = 12


def _sc_body(cf_hbm, idx_hbm, xt_hbm, yt_hbm,
             idxall, wbufF, xtbuf, ybuf, rbufG,
             sem_w0, sem_w1, sem_x0, sem_x1):
    wid = lax.axis_index("s") * 2 + lax.axis_index("c")
    base_b = wid * BPT

    pltpu.sync_copy(idx_hbm, idxall.at[pl.ds(0, B)])

    lanev = lax.iota(jnp.int32, 16)
    wrow = lanev * N
    zeros = jnp.zeros((16,), jnp.int32)
    wsems = (sem_w0, sem_w1)
    xsems = (sem_x0, sem_x1)

    def win_copy(f, b, buf):
        i = idxall[pl.ds(b, 16)][0]
        a = jnp.minimum((i // 8) * 8, P - WIN)
        return pltpu.make_async_copy(
            cf_hbm.at[pl.ds(f * N, N), pl.ds(a, WIN)],
            wbufF.at[buf], wsems[buf])

    def xt_copy(seg, buf):
        g, f = divmod(seg, 3)
        return pltpu.make_async_copy(
            xtbuf.at[buf],
            xt_hbm.at[pl.ds(f * WN, WN), pl.ds(base_b + g * GB, GB)],
            xsems[buf])

    win_copy(0, base_b, 0).start()
    for seg in range(6):
        g, f = divmod(seg, 3)
        grp = base_b + g * GB
        xb = seg % 2
        if seg >= 2:
            xt_copy(seg - 2, xb).wait()

        def pair2(t, carry, f=f, grp=grp, g=g, xb=xb):
            for k in (0, 1):
                jj = 2 * t + k
                b = grp + jj

                if k == 0:
                    win_copy(f, b + 1, 1 - k).start()
                else:
                    @pl.when(t < 7)
                    def _(f=f, b=b, k=k):
                        win_copy(f, b + 1, 1 - k).start()

                win_copy(f, b, k).wait()
                i = idxall[pl.ds(b, 16)][0]
                a = jnp.minimum((i // 8) * 8, P - WIN)
                o = i - a
                kv = zeros + k
                xbv = zeros + xb

                if f == 0:
                    for c in range(4):
                        d = plsc.load_gather(
                            wbufF, [kv, lanev + 16 * c, zeros + (o + W - 1)])
                        rbufG[jj, pl.ds(16 * c, 16)] = 1.0 / d

                colv = zeros + jj

                def per_n(n, c2, o=o, jj=jj, k=k, colv=colv, xbv=xbv):
                    rn = rbufG[jj, pl.ds(n, 16)][0]
                    v0 = wbufF[k, n, pl.ds(o, 16)] * rn
                    v1 = wbufF[k, n, pl.ds(o + 16, 16)] * rn
                    v2 = wbufF[k, n, pl.ds(o + 32, 16)] * rn
                    v3 = wbufF[k, n, pl.ds(o + W - 16, 16)] * rn
                    plsc.store_scatter(xtbuf, [xbv, wrow + n, colv], v0)
                    plsc.store_scatter(
                        xtbuf, [xbv, wrow + (16 * N + n), colv], v1)
                    plsc.store_scatter(
                        xtbuf, [xbv, wrow + (32 * N + n), colv], v2)
                    plsc.store_scatter(
                        xtbuf, [xbv, wrow + ((W - 16) * N + n), colv], v3)
                    return c2

                lax.fori_loop(0, N, per_n, 0)

                ycol = zeros + (g * GB + jj)
                for c in range(4):
                    gv = plsc.load_gather(
                        wbufF, [kv, lanev + 16 * c, zeros + (o + W)])
                    rn_v = rbufG[jj, pl.ds(16 * c, 16)]
                    plsc.store_scatter(
                        ybuf, [lanev + (f * N + 16 * c), ycol], gv * rn_v)
            return carry

        lax.fori_loop(0, 8, pair2, 0)
        if seg < 5:
            ng, nf = divmod(seg + 1, 3)
            win_copy(nf, base_b + ng * GB, 0).start()
        xt_copy(seg, xb).start()

    xt_copy(4, 0).wait()
    xt_copy(5, 1).wait()
    pltpu.sync_copy(ybuf, yt_hbm.at[:, pl.ds(base_b, BPT)])


def _sc_xy(cf2, index):
    mesh = plsc.VectorSubcoreMesh(core_axis_name="c", subcore_axis_name="s")
    fn = pl.kernel(
        _sc_body,
        out_type=[
            jax.ShapeDtypeStruct((F * WN, B), jnp.float32),
            jax.ShapeDtypeStruct((FN, B), jnp.float32),
        ],
        mesh=mesh,
        compiler_params=pltpu.CompilerParams(
            use_tc_tiling_on_sc=False, needs_layout_passes=False),
        scratch_types=[
            pltpu.VMEM((B + 16,), jnp.int32),
            pltpu.VMEM((2, N, WIN), jnp.float32),
            pltpu.VMEM((2, WN, GB), jnp.float32),
            pltpu.VMEM((FN, BPT), jnp.float32),
            pltpu.VMEM((GB, N + 16), jnp.float32),
            pltpu.SemaphoreType.DMA,
            pltpu.SemaphoreType.DMA,
            pltpu.SemaphoreType.DMA,
            pltpu.SemaphoreType.DMA,
        ],
    )
    return fn(cf2, index)


def _tc_body(idx_ref, wt_ref, pvmt_ref, outt_ref, lwt_ref,
             perm_s, st_s, perm_g, st_g, cnt, cur):
    s = pl.program_id(0)

    @pl.when(s == 0)
    def _build_routing():
        for perm, st, shift_src in ((perm_s, st_s, 0), (perm_g, st_g, 1)):
            def zero(k, c):
                cnt[k] = 0
                return c
            lax.fori_loop(0, NS, zero, 0)

            def count(b, c):
                sl = (idx_ref[b] - shift_src) >> SLAB_SHIFT
                cnt[sl] = cnt[sl] + 1
                return c
            lax.fori_loop(0, B, count, 0)

            st[0] = 0

            def prefix(k, c):
                st[k + 1] = st[k] + cnt[k]
                cur[k] = st[k]
                return c
            lax.fori_loop(0, NS, prefix, 0)

            def place(b, c):
                sl = (idx_ref[b] - shift_src) >> SLAB_SHIFT
                perm[cur[sl]] = b
                cur[sl] = cur[sl] + 1
                return c
            lax.fori_loop(0, B, place, 0)

    outt_ref[...] = pvmt_ref[...]
    base = s * SLAB

    def scat(k, c):
        b = perm_s[k]
        outt_ref[pl.ds(idx_ref[b] - base, 1), :] = wt_ref[pl.ds(b, 1), :]
        return c

    lax.fori_loop(st_s[s], st_s[s + 1], scat, 0)

    def gath(k, c):
        b = perm_g[k]
        lwt_ref[pl.ds(b, 1), :] = pvmt_ref[pl.ds(idx_ref[b] - 1 - base, 1), :]
        return c

    lax.fori_loop(st_g[s], st_g[s + 1], gath, 0)


def _tc_pvm(pvm, index, w):
    return pl.pallas_call(
        _tc_body,
        grid=(NS,),
        in_specs=[
            pl.BlockSpec(memory_space=pltpu.SMEM),
            pl.BlockSpec((B, N), lambda i: (0, 0)),
            pl.BlockSpec((SLAB, N), lambda i: (i, 0)),
        ],
        out_specs=[
            pl.BlockSpec((SLAB, N), lambda i: (i, 0)),
            pl.BlockSpec((B, N), lambda i: (0, 0)),
        ],
        out_shape=[
            jax.ShapeDtypeStruct((P, N), jnp.float32),
            jax.ShapeDtypeStruct((B, N), jnp.float32),
        ],
        scratch_shapes=[
            pltpu.SMEM((B,), jnp.int32),
            pltpu.SMEM((NS + 1,), jnp.int32),
            pltpu.SMEM((B,), jnp.int32),
            pltpu.SMEM((NS + 1,), jnp.int32),
            pltpu.SMEM((NS,), jnp.int32),
            pltpu.SMEM((NS,), jnp.int32),
        ],
        compiler_params=pltpu.CompilerParams(
            dimension_semantics=("arbitrary",)),
    )(index, w, pvm)


def kernel(coin_features, pvm, index, w):
    cf2 = coin_features.reshape(FN, P)
    xt2, yt2 = _sc_xy(cf2, index)
    new_pvm, last_w = _tc_pvm(pvm, index, w)
    X = xt2.reshape(F, W, N, B).transpose(3, 0, 2, 1)
    y = yt2.reshape(F, N, B).transpose(2, 0, 1)
    return X, y, last_w, new_pvm


# speedup vs baseline: 1.4516x; 1.1439x over previous
import jax
import jax.numpy as jnp
from jax import lax
from jax.experimental import pallas as pl
from jax.experimental.pallas import tpu as pltpu
from jax.experimental.pallas import tpu_sc as plsc

F, N, P, W, B = 3, 64, 131072, 50, 1024
FN = F * N
WIN = 64
WS = 65
GBS = 17
BPS = 33
NW = 32
BPT = B // NW
GB = 16
WN = W * N
NS = 32
SLAB = P // NS
SLAB_SHIFT = 12


def _sc_body(cf_hbm, idx_hbm, xt_hbm, yt_hbm,
             idxall, wbufF, xtbuf, ybuf, rbufG,
             sem_w0, sem_w1, sem_x0, sem_x1):
    wid = lax.axis_index("s") * 2 + lax.axis_index("c")
    base_b = wid * BPT

    pltpu.sync_copy(idx_hbm, idxall.at[pl.ds(0, B)])

    lanev = lax.iota(jnp.int32, 16)
    zeros = jnp.zeros((16,), jnp.int32)
    wsems = (sem_w0, sem_w1)
    xsems = (sem_x0, sem_x1)

    def win_copy(f, b, buf):
        i = idxall[pl.ds(b, 16)][0]
        a = jnp.minimum((i // 8) * 8, P - WIN)
        return pltpu.make_async_copy(
            cf_hbm.at[pl.ds(f * N, N), pl.ds(a, WIN)],
            wbufF.at[buf, :, pl.ds(0, WIN)], wsems[buf])

    def xt_copy(seg, buf):
        g, f = divmod(seg, 3)
        return pltpu.make_async_copy(
            xtbuf.at[0, :, :, pl.ds(0, GB)],
            xt_hbm.at[f, :, :, pl.ds(base_b + g * GB, GB)],
            xsems[0])

    win_copy(0, base_b, 0).start()
    for seg in range(6):
        g, f = divmod(seg, 3)
        grp = base_b + g * GB
        xb = 0
        if seg >= 1:
            xt_copy(seg - 1, 0).wait()

        def pair2(t, carry, f=f, grp=grp, g=g, xb=xb):
            for k in (0, 1):
                jj = 2 * t + k
                b = grp + jj

                if k == 0:
                    win_copy(f, b + 1, 1 - k).start()
                else:
                    @pl.when(t < 7)
                    def _(f=f, b=b, k=k):
                        win_copy(f, b + 1, 1 - k).start()

                win_copy(f, b, k).wait()
                i = idxall[pl.ds(b, 16)][0]
                a = jnp.minimum((i // 8) * 8, P - WIN)
                o = i - a
                kv = zeros + k
                xbv = zeros + xb
                colv = zeros + jj
                nrows = tuple(lanev + 16 * c for c in range(4))

                if f == 0:
                    for c in range(4):
                        d = plsc.load_gather(
                            wbufF, [kv, nrows[c], zeros + (o + W - 1)])
                        rbufG[jj, pl.ds(16 * c, 16)] = 1.0 / d

                rn = tuple(rbufG[jj, pl.ds(16 * c, 16)] for c in range(4))

                def per_w(w, c2, o=o, jj=jj, kv=kv, xbv=xbv, colv=colv,
                          nrows=nrows, rn=rn):
                    colw = zeros + (o + w)
                    wv = zeros + w
                    for c in range(4):
                        gv = plsc.load_gather(wbufF, [kv, nrows[c], colw])
                        plsc.store_scatter(
                            xtbuf, [xbv, wv, nrows[c], colv], gv * rn[c])
                    return c2

                lax.fori_loop(0, W, per_w, 0)

                fv = zeros + f
                ycol = zeros + (g * GB + jj)
                for c in range(4):
                    gv = plsc.load_gather(
                        wbufF, [kv, nrows[c], zeros + (o + W)])
                    plsc.store_scatter(
                        ybuf, [fv, nrows[c], ycol], gv * rn[c])
            return carry

        lax.fori_loop(0, 8, pair2, 0)
        if seg < 5:
            ng, nf = divmod(seg + 1, 3)
            win_copy(nf, base_b + ng * GB, 0).start()
        xt_copy(seg, xb).start()

    xt_copy(5, 0).wait()
    pltpu.sync_copy(ybuf.at[:, :, pl.ds(0, BPT)],
                    yt_hbm.at[:, :, pl.ds(base_b, BPT)])


def _sc_xy(cf2, index):
    mesh = plsc.VectorSubcoreMesh(core_axis_name="c", subcore_axis_name="s")
    fn = pl.kernel(
        _sc_body,
        out_type=[
            jax.ShapeDtypeStruct((F, W, N, B), jnp.float32),
            jax.ShapeDtypeStruct((F, N, B), jnp.float32),
        ],
        mesh=mesh,
        compiler_params=pltpu.CompilerParams(
            use_tc_tiling_on_sc=False, needs_layout_passes=False),
        scratch_types=[
            pltpu.VMEM((B + 16,), jnp.int32),
            pltpu.VMEM((2, N, WS), jnp.float32),
            pltpu.VMEM((1, W, N, GBS), jnp.float32),
            pltpu.VMEM((F, N, BPS), jnp.float32),
            pltpu.VMEM((GB, N + 16), jnp.float32),
            pltpu.SemaphoreType.DMA,
            pltpu.SemaphoreType.DMA,
            pltpu.SemaphoreType.DMA,
            pltpu.SemaphoreType.DMA,
        ],
    )
    return fn(cf2, index)


def _tc_body(idx_ref, wt_ref, pvmt_ref, outt_ref, lwt_ref,
             perm_s, st_s, perm_g, st_g, cnt, cur):
    s = pl.program_id(0)

    @pl.when(s == 0)
    def _build_routing():
        for perm, st, shift_src in ((perm_s, st_s, 0), (perm_g, st_g, 1)):
            def zero(k, c):
                cnt[k] = 0
                return c
            lax.fori_loop(0, NS, zero, 0)

            def count(b, c):
                sl = (idx_ref[b] - shift_src) >> SLAB_SHIFT
                cnt[sl] = cnt[sl] + 1
                return c
            lax.fori_loop(0, B, count, 0)

            st[0] = 0

            def prefix(k, c):
                st[k + 1] = st[k] + cnt[k]
                cur[k] = st[k]
                return c
            lax.fori_loop(0, NS, prefix, 0)

            def place(b, c):
                sl = (idx_ref[b] - shift_src) >> SLAB_SHIFT
                perm[cur[sl]] = b
                cur[sl] = cur[sl] + 1
                return c
            lax.fori_loop(0, B, place, 0)

    outt_ref[...] = pvmt_ref[...]
    base = s * SLAB

    def scat(k, c):
        b = perm_s[k]
        outt_ref[pl.ds(idx_ref[b] - base, 1), :] = wt_ref[pl.ds(b, 1), :]
        return c

    lax.fori_loop(st_s[s], st_s[s + 1], scat, 0)

    def gath(k, c):
        b = perm_g[k]
        lwt_ref[pl.ds(b, 1), :] = pvmt_ref[pl.ds(idx_ref[b] - 1 - base, 1), :]
        return c

    lax.fori_loop(st_g[s], st_g[s + 1], gath, 0)


def _tc_pvm(pvm, index, w):
    return pl.pallas_call(
        _tc_body,
        grid=(NS,),
        in_specs=[
            pl.BlockSpec(memory_space=pltpu.SMEM),
            pl.BlockSpec((B, N), lambda i: (0, 0)),
            pl.BlockSpec((SLAB, N), lambda i: (i, 0)),
        ],
        out_specs=[
            pl.BlockSpec((SLAB, N), lambda i: (i, 0)),
            pl.BlockSpec((B, N), lambda i: (0, 0)),
        ],
        out_shape=[
            jax.ShapeDtypeStruct((P, N), jnp.float32),
            jax.ShapeDtypeStruct((B, N), jnp.float32),
        ],
        scratch_shapes=[
            pltpu.SMEM((B,), jnp.int32),
            pltpu.SMEM((NS + 1,), jnp.int32),
            pltpu.SMEM((B,), jnp.int32),
            pltpu.SMEM((NS + 1,), jnp.int32),
            pltpu.SMEM((NS,), jnp.int32),
            pltpu.SMEM((NS,), jnp.int32),
        ],
        compiler_params=pltpu.CompilerParams(
            dimension_semantics=("arbitrary",)),
    )(index, w, pvm)


def kernel(coin_features, pvm, index, w):
    cf2 = coin_features.reshape(FN, P)
    xt4, yt3 = _sc_xy(cf2, index)
    new_pvm, last_w = _tc_pvm(pvm, index, w)
    X = xt4.transpose(3, 0, 2, 1)
    y = yt3.transpose(2, 0, 1)
    return X, y, last_w, new_pvm


# speedup vs baseline: 1.7649x; 1.2158x over previous
import jax
import jax.numpy as jnp
from jax import lax
from jax.experimental import pallas as pl
from jax.experimental.pallas import tpu as pltpu
from jax.experimental.pallas import tpu_sc as plsc

F, N, P, W, B = 3, 64, 131072, 50, 1024
FN = F * N
WIN = 64
WS = 65
GBS = 17
BPS = 33
NW = 32
BPT = B // NW
GB = 16
WN = W * N
NS = 32
SLAB = P // NS
SLAB_SHIFT = 12


def _sc_body(cf_hbm, idx_hbm, xt_hbm, yt_hbm,
             idxall, wbufF, xtbuf, ybuf, rbufG,
             sem_w0, sem_w1, sem_x0, sem_x1):
    wid = lax.axis_index("s") * 2 + lax.axis_index("c")
    base_b = wid * BPT

    pltpu.sync_copy(idx_hbm, idxall.at[pl.ds(0, B)])

    lanev = lax.iota(jnp.int32, 16)
    zeros = jnp.zeros((16,), jnp.int32)
    wsems = (sem_w0, sem_w1)
    xsems = (sem_x0, sem_x1)

    def win_copy(f, b, buf):
        i = idxall[pl.ds(b, 16)][0]
        a = jnp.minimum((i // 8) * 8, P - WIN)
        return pltpu.make_async_copy(
            cf_hbm.at[pl.ds(f * N, N), pl.ds(a, WIN)],
            wbufF.at[buf, :, pl.ds(0, WIN)], wsems[buf])

    bt0 = wid // 4
    bi0 = (wid % 4) * BPT

    def xt_copy(seg, buf):
        g, f = divmod(seg, 3)
        return pltpu.make_async_copy(
            xtbuf.at[0, :, :, :, pl.ds(0, GB)],
            xt_hbm.at[f, :, :, bt0, :, pl.ds(bi0 + g * GB, GB)],
            xsems[0])

    win_copy(0, base_b, 0).start()
    for seg in range(6):
        g, f = divmod(seg, 3)
        grp = base_b + g * GB
        xb = 0
        if seg >= 1:
            xt_copy(seg - 1, 0).wait()

        def pair2(t, carry, f=f, grp=grp, g=g, xb=xb):
            for k in (0, 1):
                jj = 2 * t + k
                b = grp + jj

                if k == 0:
                    win_copy(f, b + 1, 1 - k).start()
                else:
                    @pl.when(t < 7)
                    def _(f=f, b=b, k=k):
                        win_copy(f, b + 1, 1 - k).start()

                win_copy(f, b, k).wait()
                i = idxall[pl.ds(b, 16)][0]
                a = jnp.minimum((i // 8) * 8, P - WIN)
                o = i - a
                kv = zeros + k
                xbv = zeros + xb
                colv = zeros + jj
                nrows = tuple(lanev + 16 * c for c in range(4))
                ntv = tuple((lanev + 16 * c) // 8 for c in range(4))
                niv = lanev % 8

                if f == 0:
                    for c in range(4):
                        d = plsc.load_gather(
                            wbufF, [kv, nrows[c], zeros + (o + W - 1)])
                        rbufG[jj, pl.ds(16 * c, 16)] = 1.0 / d

                rn = tuple(rbufG[jj, pl.ds(16 * c, 16)] for c in range(4))

                def per_w(w, c2, o=o, jj=jj, kv=kv, xbv=xbv, colv=colv,
                          nrows=nrows, rn=rn, ntv=ntv, niv=niv):
                    colw = zeros + (o + w)
                    wv = zeros + w
                    for c in range(4):
                        gv = plsc.load_gather(wbufF, [kv, nrows[c], colw])
                        plsc.store_scatter(
                            xtbuf, [xbv, wv, ntv[c], niv, colv], gv * rn[c])
                    return c2

                lax.fori_loop(0, W, per_w, 0)

                fv = zeros + f
                ycol = zeros + (g * GB + jj)
                for c in range(4):
                    gv = plsc.load_gather(
                        wbufF, [kv, nrows[c], zeros + (o + W)])
                    plsc.store_scatter(
                        ybuf, [fv, nrows[c], ycol], gv * rn[c])
            return carry

        lax.fori_loop(0, 8, pair2, 0)
        if seg < 5:
            ng, nf = divmod(seg + 1, 3)
            win_copy(nf, base_b + ng * GB, 0).start()
        xt_copy(seg, xb).start()

    xt_copy(5, 0).wait()
    pltpu.sync_copy(ybuf.at[:, :, pl.ds(0, BPT)],
                    yt_hbm.at[:, :, pl.ds(base_b, BPT)])


def _sc_xy(cf2, index):
    mesh = plsc.VectorSubcoreMesh(core_axis_name="c", subcore_axis_name="s")
    fn = pl.kernel(
        _sc_body,
        out_type=[
            jax.ShapeDtypeStruct((F, W, 8, 8, 8, 128), jnp.float32),
            jax.ShapeDtypeStruct((F, N, B), jnp.float32),
        ],
        mesh=mesh,
        compiler_params=pltpu.CompilerParams(
            use_tc_tiling_on_sc=False, needs_layout_passes=False),
        scratch_types=[
            pltpu.VMEM((B + 16,), jnp.int32),
            pltpu.VMEM((2, N, WS), jnp.float32),
            pltpu.VMEM((1, W, 8, 8, GBS), jnp.float32),
            pltpu.VMEM((F, N, BPS), jnp.float32),
            pltpu.VMEM((GB, N + 16), jnp.float32),
            pltpu.SemaphoreType.DMA,
            pltpu.SemaphoreType.DMA,
            pltpu.SemaphoreType.DMA,
            pltpu.SemaphoreType.DMA,
        ],
    )
    return fn(cf2, index)


def _tc_body(idx_ref, wt_ref, pvmt_ref, outt_ref, lwt_ref,
             perm_s, st_s, perm_g, st_g, cnt, cur):
    s = pl.program_id(0)

    @pl.when(s == 0)
    def _build_routing():
        for perm, st, shift_src in ((perm_s, st_s, 0), (perm_g, st_g, 1)):
            def zero(k, c):
                cnt[k] = 0
                return c
            lax.fori_loop(0, NS, zero, 0)

            def count(b, c):
                sl = (idx_ref[b] - shift_src) >> SLAB_SHIFT
                cnt[sl] = cnt[sl] + 1
                return c
            lax.fori_loop(0, B, count, 0)

            st[0] = 0

            def prefix(k, c):
                st[k + 1] = st[k] + cnt[k]
                cur[k] = st[k]
                return c
            lax.fori_loop(0, NS, prefix, 0)

            def place(b, c):
                sl = (idx_ref[b] - shift_src) >> SLAB_SHIFT
                perm[cur[sl]] = b
                cur[sl] = cur[sl] + 1
                return c
            lax.fori_loop(0, B, place, 0)

    outt_ref[...] = pvmt_ref[...]
    base = s * SLAB

    def scat(k, c):
        b = perm_s[k]
        outt_ref[pl.ds(idx_ref[b] - base, 1), :] = wt_ref[pl.ds(b, 1), :]
        return c

    lax.fori_loop(st_s[s], st_s[s + 1], scat, 0)

    def gath(k, c):
        b = perm_g[k]
        lwt_ref[pl.ds(b, 1), :] = pvmt_ref[pl.ds(idx_ref[b] - 1 - base, 1), :]
        return c

    lax.fori_loop(st_g[s], st_g[s + 1], gath, 0)


def _tc_pvm(pvm, index, w):
    return pl.pallas_call(
        _tc_body,
        grid=(NS,),
        in_specs=[
            pl.BlockSpec(memory_space=pltpu.SMEM),
            pl.BlockSpec((B, N), lambda i: (0, 0)),
            pl.BlockSpec((SLAB, N), lambda i: (i, 0)),
        ],
        out_specs=[
            pl.BlockSpec((SLAB, N), lambda i: (i, 0)),
            pl.BlockSpec((B, N), lambda i: (0, 0)),
        ],
        out_shape=[
            jax.ShapeDtypeStruct((P, N), jnp.float32),
            jax.ShapeDtypeStruct((B, N), jnp.float32),
        ],
        scratch_shapes=[
            pltpu.SMEM((B,), jnp.int32),
            pltpu.SMEM((NS + 1,), jnp.int32),
            pltpu.SMEM((B,), jnp.int32),
            pltpu.SMEM((NS + 1,), jnp.int32),
            pltpu.SMEM((NS,), jnp.int32),
            pltpu.SMEM((NS,), jnp.int32),
        ],
        compiler_params=pltpu.CompilerParams(
            dimension_semantics=("arbitrary",)),
    )(index, w, pvm)


def kernel(coin_features, pvm, index, w):
    cf2 = coin_features.reshape(FN, P)
    xt6, yt3 = _sc_xy(cf2, index)
    new_pvm, last_w = _tc_pvm(pvm, index, w)
    X = xt6.transpose(3, 5, 0, 2, 4, 1).reshape(B, F, N, W)
    y = yt3.transpose(2, 0, 1)
    return X, y, last_w, new_pvm
